# baseline (speedup 1.0000x reference)
import functools

import jax
import jax.numpy as jnp
from jax import lax
from jax.experimental import pallas as pl
from jax.experimental.pallas import tpu as pltpu
from jax.experimental.pallas import tpu_sc as plsc

_B, _S, _D = 2, 2048, 1024
_E, _TOPK, _DFF = 8, 2, 2048
_T = _B * _S
_TM = 256
_G = _T * _TOPK
_NT = _G // _TM
_WS = _NT + _E - 1
_LANES = 128

_NC, _NS, _L = 2, 16, 16
_DW = _D // 2


def _router_body(x_ref, vr_ref, wr_ref, wsm_ref, br_ref, idx_ref, gate_ref):
    logits = jnp.dot(x_ref[...], wr_ref[...], preferred_element_type=jnp.float32)
    logits = logits + jnp.dot(vr_ref[...], wsm_ref[...],
                              preferred_element_type=jnp.float32)
    logits = logits + br_ref[...]
    lane = jax.lax.broadcasted_iota(jnp.int32, logits.shape, 1)
    logits = jnp.where(lane < _E, logits, -1e30)
    m1 = jnp.max(logits, axis=1, keepdims=True)
    i1 = jnp.min(jnp.where(logits == m1, lane, _LANES), axis=1, keepdims=True)
    rest = jnp.where(lane == i1, -1e30, logits)
    m2 = jnp.max(rest, axis=1, keepdims=True)
    i2 = jnp.min(jnp.where(rest == m2, lane, _LANES), axis=1, keepdims=True)
    w1 = 1.0 / (1.0 + jnp.exp(m2 - m1))
    w2 = 1.0 - w1
    idx_ref[...] = jnp.where(lane == 0, i1, jnp.where(lane == 1, i2, 0))
    gate_ref[...] = jnp.where(lane == 0, w1, jnp.where(lane == 1, w2, 0.0))


def _run_router(x, vr, wr_pad, wsm, br2):
    grid = (_T // _TM,)
    return pl.pallas_call(
        _router_body,
        grid=grid,
        in_specs=[
            pl.BlockSpec((_TM, _D), lambda i: (i, 0)),
            pl.BlockSpec((_TM, _LANES), lambda i: (i, 0)),
            pl.BlockSpec((_D, _LANES), lambda i: (0, 0)),
            pl.BlockSpec((_LANES, _LANES), lambda i: (0, 0)),
            pl.BlockSpec((1, _LANES), lambda i: (0, 0)),
        ],
        out_specs=[
            pl.BlockSpec((_TM, _LANES), lambda i: (i, 0)),
            pl.BlockSpec((_TM, _LANES), lambda i: (i, 0)),
        ],
        out_shape=[
            jax.ShapeDtypeStruct((_T, _LANES), jnp.int32),
            jax.ShapeDtypeStruct((_T, _LANES), jnp.float32),
        ],
    )(x, vr, wr_pad, wsm, br2)


def _ggemm_body(tid_r, eid_r, rs_r, re_r, init_r,
                x_ref, w1_ref, b1_ref, w2_ref, b2_ref, g_ref, y_ref):
    w = pl.program_id(0)
    h = jnp.dot(x_ref[...], w1_ref[0], preferred_element_type=jnp.float32)
    h = jax.nn.gelu(h + b1_ref[0])
    y = jnp.dot(h.astype(jnp.bfloat16), w2_ref[0],
                preferred_element_type=jnp.float32)
    y = (y + b2_ref[0]) * g_ref[...]
    rows = tid_r[w] * _TM + jax.lax.broadcasted_iota(jnp.int32, (_TM, 1), 0)
    contrib = jnp.where((rows >= rs_r[w]) & (rows < re_r[w]), y, 0.0)

    @pl.when(init_r[w] != 0)
    def _():
        y_ref[...] = contrib

    @pl.when(init_r[w] == 0)
    def _():
        y_ref[...] = y_ref[...] + contrib


def _run_ggemm(meta, xs, w1, b1, w2, b2, gs):
    tid, eid, rs, re, init = meta
    grid_spec = pltpu.PrefetchScalarGridSpec(
        num_scalar_prefetch=5,
        grid=(_WS,),
        in_specs=[
            pl.BlockSpec((_TM, _D), lambda w, tid, eid, rs, re, init: (tid[w], 0)),
            pl.BlockSpec((1, _D, _DFF), lambda w, tid, eid, rs, re, init: (eid[w], 0, 0)),
            pl.BlockSpec((1, 1, _DFF), lambda w, tid, eid, rs, re, init: (eid[w], 0, 0)),
            pl.BlockSpec((1, _DFF, _D), lambda w, tid, eid, rs, re, init: (eid[w], 0, 0)),
            pl.BlockSpec((1, 1, _D), lambda w, tid, eid, rs, re, init: (eid[w], 0, 0)),
            pl.BlockSpec((_TM, 1), lambda w, tid, eid, rs, re, init: (tid[w], 0)),
        ],
        out_specs=pl.BlockSpec((_TM, _D), lambda w, tid, eid, rs, re, init: (tid[w], 0)),
    )
    return pl.pallas_call(
        _ggemm_body,
        grid_spec=grid_spec,
        out_shape=jax.ShapeDtypeStruct((_G, _D), jnp.float32),
        compiler_params=pltpu.CompilerParams(
            dimension_semantics=("arbitrary",)),
    )(tid, eid, rs, re, init, xs, w1, b1, w2, b2, gs)


def _make_metadata(counts):
    off = jnp.concatenate([jnp.zeros((1,), jnp.int32),
                           jnp.cumsum(counts).astype(jnp.int32)])
    first_tile = off[:_E] // _TM
    last_tile = jnp.where(counts > 0, (off[1:] - 1) // _TM, first_tile - 1)
    num_items = last_tile - first_tile + 1
    cum = jnp.cumsum(num_items)
    total = cum[-1]
    w = jnp.arange(_WS, dtype=jnp.int32)
    eid = jnp.searchsorted(cum, w, side="right").astype(jnp.int32)
    valid = w < total
    eidc = jnp.clip(eid, 0, _E - 1)
    item_start = cum[eidc] - num_items[eidc]
    tid = first_tile[eidc] + (w - item_start)
    rs = jnp.maximum(off[eidc], tid * _TM)
    re = jnp.minimum(off[eidc + 1], (tid + 1) * _TM)
    init = ((rs == tid * _TM) & valid).astype(jnp.int32)
    tid = jnp.where(valid, tid, _NT - 1).astype(jnp.int32)
    rs = jnp.where(valid, rs, _G).astype(jnp.int32)
    re = jnp.where(valid, re, _G).astype(jnp.int32)
    eid = jnp.where(valid, eidc, _E - 1).astype(jnp.int32)
    return tid, eid, rs, re, init


_SC_PARAMS = pltpu.CompilerParams(needs_layout_passes=False,
                              use_tc_tiling_on_sc=False)

_CHUNK = _G // _NS
_TOKCH = _CHUNK // 2


def _dispatch_body(e_hbm, w_hbm, stok_hbm, sgate_hbm, dest_hbm, counts_hbm,
                   e_buf, w_buf, tok_buf, pos_buf, dest_buf,
                   hist_v, hist_all, hist_sh, stok_sh, sgate_sh):
    sid = lax.axis_index("s")
    chunk0 = sid * _CHUNK
    lanes = jnp.arange(_L, dtype=jnp.int32)
    zeros = jnp.zeros((_L,), jnp.int32)
    pltpu.sync_copy(e_hbm.at[pl.ds(chunk0, _CHUNK)], e_buf)
    pltpu.sync_copy(w_hbm.at[pl.ds(chunk0, _CHUNK)], w_buf)

    def hstep(v, hist):
        ev = e_buf[pl.ds(v * _L, _L)]
        for e in range(_E):
            c = plsc.all_reduce_population_count(ev == e)
            hist = hist + jnp.where(lanes == e, c, zeros)
        return hist
    hist = lax.fori_loop(0, _CHUNK // _L, hstep, zeros)
    hist_v[...] = hist
    pltpu.sync_copy(hist_v, hist_sh.at[sid])
    plsc.subcore_barrier()

    pltpu.sync_copy(hist_sh, hist_all)

    def bstep(s, carry):
        run, prior = carry
        row = hist_all[s]
        flag = (s < sid).astype(jnp.int32)
        return run + row, prior + row * flag
    run, prior = lax.fori_loop(0, _NS, bstep, (zeros, zeros))
    base = plsc.cumsum(run) - run + prior

    @pl.when(sid == 0)
    def _():
        hist_v[...] = run
        pltpu.sync_copy(hist_v, counts_hbm)

    def cstep(v, cnt):
        ev = e_buf[pl.ds(v * _L, _L)]
        jv = chunk0 + v * _L + lanes
        mybase = jnp.take(base, ev)
        myprev = jnp.take(cnt, ev)
        rank = zeros
        for e in range(_E):
            m = ev == e
            cs = plsc.cumsum(m.astype(jnp.int32))
            rank = jnp.where(m, cs - 1, rank)
            c = plsc.all_reduce_population_count(m)
            cnt = cnt + jnp.where(lanes == e, c, zeros)
        pos = mybase + myprev + rank
        pos_buf[pl.ds(v * _L, _L)] = pos
        tok_buf[pl.ds(v * _L, _L)] = jv >> 1
        addr = (jv & 1) * _TOKCH + ((jv - chunk0) >> 1)
        plsc.store_scatter(dest_buf, [addr], pos)
        return cnt
    lax.fori_loop(0, _CHUNK // _L, cstep, zeros)

    pltpu.sync_copy(tok_buf, stok_sh.at[pos_buf])
    pltpu.sync_copy(w_buf, sgate_sh.at[pos_buf])
    plsc.subcore_barrier()
    pltpu.sync_copy(stok_sh.at[pl.ds(chunk0, _CHUNK)], tok_buf)
    pltpu.sync_copy(tok_buf, stok_hbm.at[pl.ds(chunk0, _CHUNK)])
    pltpu.sync_copy(sgate_sh.at[pl.ds(chunk0, _CHUNK)], w_buf)
    pltpu.sync_copy(w_buf, sgate_hbm.at[pl.ds(chunk0, _CHUNK)])
    tokbase = sid * _TOKCH
    pltpu.sync_copy(dest_buf.at[pl.ds(0, _TOKCH)],
                    dest_hbm.at[0, pl.ds(tokbase, _TOKCH)])
    pltpu.sync_copy(dest_buf.at[pl.ds(_TOKCH, _TOKCH)],
                    dest_hbm.at[1, pl.ds(tokbase, _TOKCH)])


def _run_dispatch(flat_e, flat_w):
    mesh = plsc.VectorSubcoreMesh(core_axis_name="c", subcore_axis_name="s",
                                  num_cores=1, num_subcores=_NS)
    f = pl.kernel(
        _dispatch_body,
        compiler_params=_SC_PARAMS,
        out_type=[
            jax.ShapeDtypeStruct((_G,), jnp.int32),
            jax.ShapeDtypeStruct((_G,), jnp.float32),
            jax.ShapeDtypeStruct((2, _T), jnp.int32),
            jax.ShapeDtypeStruct((_L,), jnp.int32),
        ],
        mesh=mesh,
        scratch_types=[
            pltpu.VMEM((_CHUNK,), jnp.int32),
            pltpu.VMEM((_CHUNK,), jnp.float32),
            pltpu.VMEM((_CHUNK,), jnp.int32),
            pltpu.VMEM((_CHUNK,), jnp.int32),
            pltpu.VMEM((_CHUNK,), jnp.int32),
            pltpu.VMEM((_L,), jnp.int32),
            pltpu.VMEM((_NS, _L), jnp.int32),
            pltpu.VMEM_SHARED((_NS, _L), jnp.int32),
            pltpu.VMEM_SHARED((_G,), jnp.int32),
            pltpu.VMEM_SHARED((_G,), jnp.float32),
        ],
    )
    return f(flat_e, flat_w)


_XROWS = _G // (_NC * _NS)
_XCH = 128


def _xgather_body(stok_hbm, xb_hbm, xs_hbm, idx_buf, rows, sem):
    wid = lax.axis_index("s") * _NC + lax.axis_index("c")
    for h in range(_XROWS // _XCH):
        rb = wid * _XROWS + h * _XCH
        pltpu.sync_copy(stok_hbm.at[pl.ds(rb, _XCH)], idx_buf)
        pltpu.async_copy(xb_hbm.at[idx_buf], rows, sem).wait()
        pltpu.sync_copy(rows, xs_hbm.at[pl.ds(rb, _XCH), :])


def _run_xgather(stok, x_bits):
    mesh = plsc.VectorSubcoreMesh(core_axis_name="c", subcore_axis_name="s",
                                  num_cores=_NC, num_subcores=_NS)
    f = pl.kernel(
        _xgather_body,
        compiler_params=_SC_PARAMS,
        out_type=jax.ShapeDtypeStruct((_G, _DW), jnp.int32),
        mesh=mesh,
        scratch_types=[
            pltpu.VMEM((_XCH,), jnp.int32),
            pltpu.VMEM((_XCH, _DW), jnp.int32),
            pltpu.SemaphoreType.DMA,
        ],
    )
    return f(stok, x_bits)


_TOKW = _T // (_NC * _NS)
_CCH = 64


def _combine_body(dest_hbm, ys_hbm, out_hbm, p_buf, acc, sem):
    wid = lax.axis_index("s") * _NC + lax.axis_index("c")
    for h in range(_TOKW // _CCH):
        tb = wid * _TOKW + h * _CCH
        pltpu.sync_copy(dest_hbm.at[0, pl.ds(tb, _CCH)], p_buf)
        pltpu.async_copy(ys_hbm.at[p_buf], acc, sem).wait()
        pltpu.sync_copy(dest_hbm.at[1, pl.ds(tb, _CCH)], p_buf)
        pltpu.async_copy(ys_hbm.at[p_buf], acc, sem, add=True).wait()
        pltpu.sync_copy(acc, out_hbm.at[pl.ds(tb, _CCH), :])


def _run_combine(dest, ys):
    mesh = plsc.VectorSubcoreMesh(core_axis_name="c", subcore_axis_name="s",
                                  num_cores=_NC, num_subcores=_NS)
    f = pl.kernel(
        _combine_body,
        compiler_params=_SC_PARAMS,
        out_type=jax.ShapeDtypeStruct((_T, _D), jnp.float32),
        mesh=mesh,
        scratch_types=[
            pltpu.VMEM((_CCH,), jnp.int32),
            pltpu.VMEM((_CCH, _D), jnp.float32),
            pltpu.SemaphoreType.DMA,
        ],
    )
    return f(dest, ys)


def kernel(embeddings, volatility, risk_factors, Wr, br, W1, b1, W2, b2):
    x = embeddings.reshape(_T, _D)
    vol = volatility.reshape(_T, 1)
    risk = risk_factors.reshape(_T, 1)

    vr = jnp.zeros((_T, _LANES), jnp.float32).at[:, 0:1].set(vol).at[:, 1:2].set(risk)
    wr_pad = jnp.zeros((_D, _LANES), jnp.float32).at[:, :_E].set(Wr[:_D])
    wsm = jnp.zeros((_LANES, _LANES), jnp.float32).at[0, :_E].set(Wr[_D]).at[1, :_E].set(Wr[_D + 1])
    br2 = jnp.zeros((1, _LANES), jnp.float32).at[0, :_E].set(br)

    idx_out, gate_out = _run_router(x, vr, wr_pad, wsm, br2)
    flat_e = idx_out[:, :_TOPK].reshape(-1)
    flat_w = gate_out[:, :_TOPK].reshape(-1)

    stok, sgate, dest, counts_v = _run_dispatch(flat_e, flat_w)
    counts = counts_v[:_E]
    meta = _make_metadata(counts)

    x_bits = lax.bitcast_convert_type(
        x.astype(jnp.bfloat16).reshape(_T, _DW, 2), jnp.int32)
    xs_bits = _run_xgather(stok, x_bits)
    xs = lax.bitcast_convert_type(xs_bits, jnp.bfloat16).reshape(_G, _D)
    gs = sgate.reshape(_G, 1)

    ys = _run_ggemm(meta, xs, W1.astype(jnp.bfloat16), b1.reshape(_E, 1, _DFF),
                    W2.astype(jnp.bfloat16), b2.reshape(_E, 1, _D), gs)

    out = _run_combine(dest, ys)
    return out.reshape(_B, _S, _D)


# speedup vs baseline: 1.0660x; 1.0660x over previous
import functools

import jax
import jax.numpy as jnp
from jax import lax
from jax.experimental import pallas as pl
from jax.experimental.pallas import tpu as pltpu
from jax.experimental.pallas import tpu_sc as plsc

_B, _S, _D = 2, 2048, 1024
_E, _TOPK, _DFF = 8, 2, 2048
_T = _B * _S
_TM = 256
_G = _T * _TOPK
_NT = _G // _TM
_WS = _NT + _E - 1
_LANES = 128

_NC, _NS, _L = 2, 16, 16
_DW = _D // 2


def _router_body(x_ref, vr_ref, wr_ref, wsm_ref, br_ref, idx_ref, gate_ref):
    logits = jnp.dot(x_ref[...], wr_ref[...], preferred_element_type=jnp.float32)
    logits = logits + jnp.dot(vr_ref[...], wsm_ref[...],
                              preferred_element_type=jnp.float32)
    logits = logits + br_ref[...]
    lane = jax.lax.broadcasted_iota(jnp.int32, logits.shape, 1)
    logits = jnp.where(lane < _E, logits, -1e30)
    m1 = jnp.max(logits, axis=1, keepdims=True)
    i1 = jnp.min(jnp.where(logits == m1, lane, _LANES), axis=1, keepdims=True)
    rest = jnp.where(lane == i1, -1e30, logits)
    m2 = jnp.max(rest, axis=1, keepdims=True)
    i2 = jnp.min(jnp.where(rest == m2, lane, _LANES), axis=1, keepdims=True)
    w1 = 1.0 / (1.0 + jnp.exp(m2 - m1))
    w2 = 1.0 - w1
    idx_ref[...] = jnp.where(lane == 0, i1, jnp.where(lane == 1, i2, 0))
    gate_ref[...] = jnp.where(lane == 0, w1, jnp.where(lane == 1, w2, 0.0))


def _run_router(x, vr, wr_pad, wsm, br2):
    grid = (_T // _TM,)
    return pl.pallas_call(
        _router_body,
        grid=grid,
        in_specs=[
            pl.BlockSpec((_TM, _D), lambda i: (i, 0)),
            pl.BlockSpec((_TM, _LANES), lambda i: (i, 0)),
            pl.BlockSpec((_D, _LANES), lambda i: (0, 0)),
            pl.BlockSpec((_LANES, _LANES), lambda i: (0, 0)),
            pl.BlockSpec((1, _LANES), lambda i: (0, 0)),
        ],
        out_specs=[
            pl.BlockSpec((_TM, _LANES), lambda i: (i, 0)),
            pl.BlockSpec((_TM, _LANES), lambda i: (i, 0)),
        ],
        out_shape=[
            jax.ShapeDtypeStruct((_T, _LANES), jnp.int32),
            jax.ShapeDtypeStruct((_T, _LANES), jnp.float32),
        ],
    )(x, vr, wr_pad, wsm, br2)


def _ggemm_body(tid_r, eid_r, rs_r, re_r, init_r,
                x_ref, w1_ref, b1_ref, w2_ref, b2_ref, g_ref, y_ref):
    w = pl.program_id(0)
    h = jnp.dot(x_ref[...], w1_ref[0], preferred_element_type=jnp.float32)
    h = jax.nn.gelu(h + b1_ref[0])
    y = jnp.dot(h.astype(jnp.bfloat16), w2_ref[0],
                preferred_element_type=jnp.float32)
    y = (y + b2_ref[0]) * g_ref[...]
    rows = tid_r[w] * _TM + jax.lax.broadcasted_iota(jnp.int32, (_TM, 1), 0)
    contrib = jnp.where((rows >= rs_r[w]) & (rows < re_r[w]), y, 0.0)

    @pl.when(init_r[w] != 0)
    def _():
        y_ref[...] = contrib

    @pl.when(init_r[w] == 0)
    def _():
        y_ref[...] = y_ref[...] + contrib


def _run_ggemm(meta, xs, w1, b1, w2, b2, gs):
    tid, eid, rs, re, init = meta
    grid_spec = pltpu.PrefetchScalarGridSpec(
        num_scalar_prefetch=5,
        grid=(_WS,),
        in_specs=[
            pl.BlockSpec((_TM, _D), lambda w, tid, eid, rs, re, init: (tid[w], 0)),
            pl.BlockSpec((1, _D, _DFF), lambda w, tid, eid, rs, re, init: (eid[w], 0, 0)),
            pl.BlockSpec((1, 1, _DFF), lambda w, tid, eid, rs, re, init: (eid[w], 0, 0)),
            pl.BlockSpec((1, _DFF, _D), lambda w, tid, eid, rs, re, init: (eid[w], 0, 0)),
            pl.BlockSpec((1, 1, _D), lambda w, tid, eid, rs, re, init: (eid[w], 0, 0)),
            pl.BlockSpec((_TM, 1), lambda w, tid, eid, rs, re, init: (tid[w], 0)),
        ],
        out_specs=pl.BlockSpec((_TM, _D), lambda w, tid, eid, rs, re, init: (tid[w], 0)),
    )
    return pl.pallas_call(
        _ggemm_body,
        grid_spec=grid_spec,
        out_shape=jax.ShapeDtypeStruct((_G, _D), jnp.float32),
        compiler_params=pltpu.CompilerParams(
            dimension_semantics=("arbitrary",)),
    )(tid, eid, rs, re, init, xs, w1, b1, w2, b2, gs)


def _make_metadata(counts):
    off = jnp.concatenate([jnp.zeros((1,), jnp.int32),
                           jnp.cumsum(counts).astype(jnp.int32)])
    first_tile = off[:_E] // _TM
    last_tile = jnp.where(counts > 0, (off[1:] - 1) // _TM, first_tile - 1)
    num_items = last_tile - first_tile + 1
    cum = jnp.cumsum(num_items)
    total = cum[-1]
    w = jnp.arange(_WS, dtype=jnp.int32)
    eid = jnp.searchsorted(cum, w, side="right").astype(jnp.int32)
    valid = w < total
    eidc = jnp.clip(eid, 0, _E - 1)
    item_start = cum[eidc] - num_items[eidc]
    tid = first_tile[eidc] + (w - item_start)
    rs = jnp.maximum(off[eidc], tid * _TM)
    re = jnp.minimum(off[eidc + 1], (tid + 1) * _TM)
    init = ((rs == tid * _TM) & valid).astype(jnp.int32)
    tid = jnp.where(valid, tid, _NT - 1).astype(jnp.int32)
    rs = jnp.where(valid, rs, _G).astype(jnp.int32)
    re = jnp.where(valid, re, _G).astype(jnp.int32)
    eid = jnp.where(valid, eidc, _E - 1).astype(jnp.int32)
    return tid, eid, rs, re, init


_SC_PARAMS = pltpu.CompilerParams(needs_layout_passes=False)

_CHUNK = _G // _NS
_TOKCH = _CHUNK // 2


def _dispatch_body(e_hbm, w_hbm, stok_hbm, sgate_hbm, dest0_hbm, dest1_hbm,
                   counts_hbm,
                   e_buf, w_buf, tok_buf, pos_buf, dest_buf,
                   hist_v, hist_all, hist_sh, stok_sh, sgate_sh):
    sid = lax.axis_index("s")
    chunk0 = sid * _CHUNK
    lanes = jnp.arange(_L, dtype=jnp.int32)
    zeros = jnp.zeros((_L,), jnp.int32)
    pltpu.sync_copy(e_hbm.at[pl.ds(chunk0, _CHUNK)], e_buf)
    pltpu.sync_copy(w_hbm.at[pl.ds(chunk0, _CHUNK)], w_buf)

    def hstep(v, hist):
        ev = e_buf[pl.ds(v * _L, _L)]
        for e in range(_E):
            c = plsc.all_reduce_population_count(ev == e)
            hist = hist + jnp.where(lanes == e, c, zeros)
        return hist
    hist = lax.fori_loop(0, _CHUNK // _L, hstep, zeros)
    hist_v[...] = hist
    pltpu.sync_copy(hist_v, hist_sh.at[sid])
    plsc.subcore_barrier()

    pltpu.sync_copy(hist_sh, hist_all)

    def bstep(s, carry):
        run, prior = carry
        row = hist_all[s]
        flag = (s < sid).astype(jnp.int32)
        return run + row, prior + row * flag
    run, prior = lax.fori_loop(0, _NS, bstep, (zeros, zeros))
    base = plsc.cumsum(run) - run + prior

    @pl.when(sid == 0)
    def _():
        hist_v[...] = run
        pltpu.sync_copy(hist_v, counts_hbm)

    def cstep(v, cnt):
        ev = e_buf[pl.ds(v * _L, _L)]
        jv = chunk0 + v * _L + lanes
        mybase = jnp.take(base, ev)
        myprev = jnp.take(cnt, ev)
        rank = zeros
        for e in range(_E):
            m = ev == e
            cs = plsc.cumsum(m.astype(jnp.int32))
            rank = jnp.where(m, cs - 1, rank)
            c = plsc.all_reduce_population_count(m)
            cnt = cnt + jnp.where(lanes == e, c, zeros)
        pos = mybase + myprev + rank
        pos_buf[pl.ds(v * _L, _L)] = pos
        tok_buf[pl.ds(v * _L, _L)] = jv >> 1
        addr = (jv & 1) * _TOKCH + ((jv - chunk0) >> 1)
        plsc.store_scatter(dest_buf, [addr], pos)
        return cnt
    lax.fori_loop(0, _CHUNK // _L, cstep, zeros)

    pltpu.sync_copy(tok_buf, stok_sh.at[pos_buf])
    pltpu.sync_copy(w_buf, sgate_sh.at[pos_buf])
    plsc.subcore_barrier()
    pltpu.sync_copy(stok_sh.at[pl.ds(chunk0, _CHUNK)], tok_buf)
    pltpu.sync_copy(tok_buf, stok_hbm.at[pl.ds(chunk0, _CHUNK)])
    pltpu.sync_copy(sgate_sh.at[pl.ds(chunk0, _CHUNK)], w_buf)
    pltpu.sync_copy(w_buf, sgate_hbm.at[pl.ds(chunk0, _CHUNK)])
    tokbase = sid * _TOKCH
    pltpu.sync_copy(dest_buf.at[pl.ds(0, _TOKCH)],
                    dest0_hbm.at[pl.ds(tokbase, _TOKCH)])
    pltpu.sync_copy(dest_buf.at[pl.ds(_TOKCH, _TOKCH)],
                    dest1_hbm.at[pl.ds(tokbase, _TOKCH)])


def _run_dispatch(flat_e, flat_w):
    mesh = plsc.VectorSubcoreMesh(core_axis_name="c", subcore_axis_name="s",
                                  num_cores=1, num_subcores=_NS)
    f = pl.kernel(
        _dispatch_body,
        compiler_params=_SC_PARAMS,
        out_type=[
            jax.ShapeDtypeStruct((_G,), jnp.int32),
            jax.ShapeDtypeStruct((_G,), jnp.float32),
            jax.ShapeDtypeStruct((_T,), jnp.int32),
            jax.ShapeDtypeStruct((_T,), jnp.int32),
            jax.ShapeDtypeStruct((_L,), jnp.int32),
        ],
        mesh=mesh,
        scratch_types=[
            pltpu.VMEM((_CHUNK,), jnp.int32),
            pltpu.VMEM((_CHUNK,), jnp.float32),
            pltpu.VMEM((_CHUNK,), jnp.int32),
            pltpu.VMEM((_CHUNK,), jnp.int32),
            pltpu.VMEM((_CHUNK,), jnp.int32),
            pltpu.VMEM((_L,), jnp.int32),
            pltpu.VMEM((_NS, _L), jnp.int32),
            pltpu.VMEM_SHARED((_NS, _L), jnp.int32),
            pltpu.VMEM_SHARED((_G,), jnp.int32),
            pltpu.VMEM_SHARED((_G,), jnp.float32),
        ],
    )
    return f(flat_e, flat_w)


_XROWS = _G // (_NC * _NS)
_XCH = 128


def _xgather_body(stok_hbm, xb_hbm, xs_hbm, idx_buf, rows, sem):
    wid = lax.axis_index("s") * _NC + lax.axis_index("c")
    for h in range(_XROWS // _XCH):
        rb = wid * _XROWS + h * _XCH
        pltpu.sync_copy(stok_hbm.at[pl.ds(rb, _XCH)], idx_buf)
        pltpu.async_copy(xb_hbm.at[idx_buf], rows, sem).wait()
        pltpu.sync_copy(rows, xs_hbm.at[pl.ds(rb, _XCH), :])


def _run_xgather(stok, x_bits):
    mesh = plsc.VectorSubcoreMesh(core_axis_name="c", subcore_axis_name="s",
                                  num_cores=_NC, num_subcores=_NS)
    f = pl.kernel(
        _xgather_body,
        out_type=jax.ShapeDtypeStruct((_G, _DW), jnp.int32),
        mesh=mesh,
        scratch_types=[
            pltpu.VMEM((_XCH,), jnp.int32),
            pltpu.VMEM((_XCH, _DW), jnp.int32),
            pltpu.SemaphoreType.DMA,
        ],
    )
    return f(stok, x_bits)


_TOKW = _T // (_NC * _NS)
_CCH = 32


def _combine_body(dest0_hbm, dest1_hbm, ys_hbm, out_hbm, p_buf, acc_a, acc_b, sem):
    wid = lax.axis_index("s") * _NC + lax.axis_index("c")
    for h in range(_TOKW // _CCH):
        tb = wid * _TOKW + h * _CCH
        pltpu.sync_copy(dest0_hbm.at[pl.ds(tb, _CCH)], p_buf)
        ca = pltpu.async_copy(ys_hbm.at[p_buf], acc_a, sem)
        ca.wait()
        pltpu.sync_copy(dest1_hbm.at[pl.ds(tb, _CCH)], p_buf)
        pltpu.async_copy(ys_hbm.at[p_buf], acc_b, sem).wait()

        def rstep(r, _):
            def cstep(c, __):
                sl = pl.ds(c * _L, _L)
                acc_a[r, sl] = acc_a[r, sl] + acc_b[r, sl]
                return 0
            return lax.fori_loop(0, _D // _L, cstep, 0)
        lax.fori_loop(0, _CCH, rstep, 0)
        pltpu.sync_copy(acc_a, out_hbm.at[pl.ds(tb, _CCH), :])


def _run_combine(dest0, dest1, ys):
    mesh = plsc.VectorSubcoreMesh(core_axis_name="c", subcore_axis_name="s",
                                  num_cores=_NC, num_subcores=_NS)
    f = pl.kernel(
        _combine_body,
        out_type=jax.ShapeDtypeStruct((_T, _D), jnp.float32),
        mesh=mesh,
        scratch_types=[
            pltpu.VMEM((_CCH,), jnp.int32),
            pltpu.VMEM((_CCH, _D), jnp.float32),
            pltpu.VMEM((_CCH, _D), jnp.float32),
            pltpu.SemaphoreType.DMA,
        ],
    )
    return f(dest0, dest1, ys)


def kernel(embeddings, volatility, risk_factors, Wr, br, W1, b1, W2, b2):
    x = embeddings.reshape(_T, _D)
    vol = volatility.reshape(_T, 1)
    risk = risk_factors.reshape(_T, 1)

    vr = jnp.zeros((_T, _LANES), jnp.float32).at[:, 0:1].set(vol).at[:, 1:2].set(risk)
    wr_pad = jnp.zeros((_D, _LANES), jnp.float32).at[:, :_E].set(Wr[:_D])
    wsm = jnp.zeros((_LANES, _LANES), jnp.float32).at[0, :_E].set(Wr[_D]).at[1, :_E].set(Wr[_D + 1])
    br2 = jnp.zeros((1, _LANES), jnp.float32).at[0, :_E].set(br)

    idx_out, gate_out = _run_router(x, vr, wr_pad, wsm, br2)
    flat_e = idx_out[:, :_TOPK].reshape(-1)
    flat_w = gate_out[:, :_TOPK].reshape(-1)

    stok, sgate, dest0, dest1, counts_v = _run_dispatch(flat_e, flat_w)
    counts = counts_v[:_E]
    meta = _make_metadata(counts)

    x_bits = lax.bitcast_convert_type(
        x.astype(jnp.bfloat16).reshape(_T, _DW, 2), jnp.int32)
    xs_bits = _run_xgather(stok, x_bits)
    xs = lax.bitcast_convert_type(xs_bits, jnp.bfloat16).reshape(_G, _D)
    gs = sgate.reshape(_G, 1)

    ys = _run_ggemm(meta, xs, W1.astype(jnp.bfloat16), b1.reshape(_E, 1, _DFF),
                    W2.astype(jnp.bfloat16), b2.reshape(_E, 1, _D), gs)

    out = _run_combine(dest0, dest1, ys)
    return out.reshape(_B, _S, _D)
